# baseline (speedup 1.0000x reference)
import jax
import jax.numpy as jnp
from jax import lax
from jax.experimental import pallas as pl
from jax.experimental.pallas import tpu as pltpu
from jax.experimental.pallas import tpu_sc as plsc

_N = 10000
_E = 320000
_D = 128
_H = 128

_NC = 2
_NS = 16
_RA = 624
_RL = _N - 15 * _RA
_EPC = _E // _NC
_EPT = _EPC // _NS

_KA = 2000
_KB = 200
_NCHB = _E // _NS // _KB
_EPTB = _E // _NS
_KC = 1000
_NCHC = _EPT // _KC

_BLK = 1000
_GRID = _N // _BLK


def _sc_mesh():
    return plsc.VectorSubcoreMesh(core_axis_name="c", subcore_axis_name="s",
                                  num_cores=_NC, num_subcores=_NS)


def _tile_slab_copy(sid, copy_fn):
    @pl.when(sid < _NS - 1)
    def _body():
        copy_fn(sid * _RA, _RA)

    @pl.when(sid == _NS - 1)
    def _last():
        copy_fn((_NS - 1) * _RA, _RL)


def _deg_body(ei_h, ones_h, zero_h, out_h, acc, idx_d, ones_v):
    cid = lax.axis_index("c")
    sid = lax.axis_index("s")
    _tile_slab_copy(sid, lambda rb, nr: pltpu.sync_copy(
        zero_h.at[pl.ds(rb, nr), :], acc.at[pl.ds(rb, nr), :]))
    pltpu.sync_copy(ones_h, ones_v)
    plsc.subcore_barrier()
    ebase = cid * _EPC + sid * _EPT

    def chunk(i, carry):
        off = ebase + i * _KA
        pltpu.sync_copy(ei_h.at[1, pl.ds(off, _KA)], idx_d)
        pltpu.sync_copy(ones_v, acc.at[idx_d], add=True)
        return carry

    lax.fori_loop(0, _EPT // _KA, chunk, 0)
    plsc.subcore_barrier()
    _tile_slab_copy(sid, lambda rb, nr: pltpu.sync_copy(
        acc.at[pl.ds(rb, nr), :], out_h.at[cid, pl.ds(rb, nr), :]))


_SC_PARAMS = pltpu.CompilerParams(use_tc_tiling_on_sc=False)

_deg_kernel = pl.kernel(
    _deg_body,
    out_type=jax.ShapeDtypeStruct((_NC, _N, 8), jnp.float32),
    mesh=_sc_mesh(),
    compiler_params=_SC_PARAMS,
    scratch_types=[
        pltpu.VMEM_SHARED((_N, 8), jnp.float32),
        pltpu.VMEM((_KA,), jnp.int32),
        pltpu.VMEM((_KA, 8), jnp.float32),
    ],
)


def _pipelined_edge_loop(ei_h, k, nch, ebase, gather_view_fn, acc,
                         idx_s, idx_d, rows, isem, gsem, ssem):
    for j in range(2):
        pltpu.async_copy(ei_h.at[0, pl.ds(ebase + j * k, k)], idx_s.at[j],
                         isem)
        pltpu.async_copy(ei_h.at[1, pl.ds(ebase + j * k, k)], idx_d.at[j],
                         isem)

    def chunk(i, carry):
        b3 = lax.rem(i, 3)
        b4 = lax.rem(i, 4)
        off = ebase + i * k
        pltpu.make_async_copy(ei_h.at[0, pl.ds(off, k)], idx_s.at[b4],
                              isem).wait()
        pltpu.make_async_copy(ei_h.at[1, pl.ds(off, k)], idx_d.at[b4],
                              isem).wait()
        gd = pltpu.async_copy(gather_view_fn(idx_s.at[b4]), rows.at[b3],
                              gsem)

        @pl.when(i >= 2)
        def _wait_scatter_i_minus_2():
            pltpu.make_async_copy(rows.at[lax.rem(i - 2, 3)],
                                  acc.at[idx_d.at[lax.rem(i - 2, 4)]],
                                  ssem).wait()

        @pl.when(i + 2 < nch)
        def _prefetch_idx():
            nb4 = lax.rem(i + 2, 4)
            pltpu.async_copy(ei_h.at[0, pl.ds(off + 2 * k, k)],
                             idx_s.at[nb4], isem)
            pltpu.async_copy(ei_h.at[1, pl.ds(off + 2 * k, k)],
                             idx_d.at[nb4], isem)

        gd.wait()
        pltpu.async_copy(rows.at[b3], acc.at[idx_d.at[b4]], ssem, add=True)
        return carry

    lax.fori_loop(0, nch, chunk, 0)
    for j in (nch - 2, nch - 1):
        pltpu.make_async_copy(rows.at[j % 3], acc.at[idx_d.at[j % 4]],
                              ssem).wait()


def _agg64_body(ei_h, tab_h, zero_h, out_h, acc, idx_s, idx_d,
                rows, isem, gsem, ssem):
    cid = lax.axis_index("c")
    sid = lax.axis_index("s")
    _tile_slab_copy(sid, lambda rb, nr: pltpu.sync_copy(
        zero_h.at[pl.ds(rb, nr), :], acc.at[pl.ds(rb, nr), :]))
    plsc.subcore_barrier()
    _pipelined_edge_loop(ei_h, _KB, _NCHB, sid * _EPTB,
                         lambda idx: tab_h.at[cid].at[idx], acc,
                         idx_s, idx_d, rows, isem, gsem, ssem)
    plsc.subcore_barrier()
    _tile_slab_copy(sid, lambda rb, nr: pltpu.sync_copy(
        acc.at[pl.ds(rb, nr), :], out_h.at[cid, pl.ds(rb, nr), :]))


_agg64 = pl.kernel(
    _agg64_body,
    out_type=jax.ShapeDtypeStruct((_NC, _N, 64), jnp.float32),
    mesh=_sc_mesh(),
    compiler_params=_SC_PARAMS,
    scratch_types=[
        pltpu.VMEM_SHARED((_N, 64), jnp.float32),
        pltpu.VMEM((4, _KB), jnp.int32),
        pltpu.VMEM((4, _KB), jnp.int32),
        pltpu.VMEM((3, _KB, 64), jnp.float32),
        pltpu.SemaphoreType.DMA,
        pltpu.SemaphoreType.DMA,
        pltpu.SemaphoreType.DMA,
    ],
)


def _agg8_body(ei_h, tab_h, zero_h, out_h, acc, idx_s, idx_d,
               rows, isem, gsem, ssem):
    cid = lax.axis_index("c")
    sid = lax.axis_index("s")
    _tile_slab_copy(sid, lambda rb, nr: pltpu.sync_copy(
        zero_h.at[pl.ds(rb, nr), :], acc.at[pl.ds(rb, nr), :]))
    plsc.subcore_barrier()
    _pipelined_edge_loop(ei_h, _KC, _NCHC, cid * _EPC + sid * _EPT,
                         lambda idx: tab_h.at[idx], acc,
                         idx_s, idx_d, rows, isem, gsem, ssem)
    plsc.subcore_barrier()
    _tile_slab_copy(sid, lambda rb, nr: pltpu.sync_copy(
        acc.at[pl.ds(rb, nr), :], out_h.at[cid, pl.ds(rb, nr), :]))


_agg8 = pl.kernel(
    _agg8_body,
    out_type=jax.ShapeDtypeStruct((_NC, _N, 8), jnp.float32),
    mesh=_sc_mesh(),
    compiler_params=_SC_PARAMS,
    scratch_types=[
        pltpu.VMEM_SHARED((_N, 8), jnp.float32),
        pltpu.VMEM((4, _KC), jnp.int32),
        pltpu.VMEM((4, _KC), jnp.int32),
        pltpu.VMEM((3, _KC, 8), jnp.float32),
        pltpu.SemaphoreType.DMA,
        pltpu.SemaphoreType.DMA,
        pltpu.SemaphoreType.DMA,
    ],
)


def _mm_scale_body(x_ref, w_ref, degp_ref, h_ref, g_ref, dinv_ref):
    h = jnp.dot(x_ref[...], w_ref[...], preferred_element_type=jnp.float32)
    deg = degp_ref[0] + degp_ref[1] + 1.0
    dinv = lax.rsqrt(deg)
    h_ref[...] = h
    g = h * dinv[:, 0:1]
    g_ref[...] = jnp.stack([g[:, :64], g[:, 64:]])
    dinv_ref[...] = dinv


_mm_scale = pl.pallas_call(
    _mm_scale_body,
    grid=(_GRID,),
    in_specs=[
        pl.BlockSpec((_BLK, _D), lambda i: (i, 0)),
        pl.BlockSpec((_D, _H), lambda i: (0, 0)),
        pl.BlockSpec((_NC, _BLK, 8), lambda i: (0, i, 0)),
    ],
    out_specs=[
        pl.BlockSpec((_BLK, _H), lambda i: (i, 0)),
        pl.BlockSpec((_NC, _BLK, 64), lambda i: (0, i, 0)),
        pl.BlockSpec((_BLK, 8), lambda i: (i, 0)),
    ],
    out_shape=[
        jax.ShapeDtypeStruct((_N, _H), jnp.float32),
        jax.ShapeDtypeStruct((_NC, _N, 64), jnp.float32),
        jax.ShapeDtypeStruct((_N, 8), jnp.float32),
    ],
)


def _bn_fused_body(z1p_ref, h_ref, dinv_ref, gamma_ref, beta_ref, w2_ref,
                   y_ref, g2_ref, y1_vmem, s1_ref, s2_ref):
    p = pl.program_id(0)
    i = pl.program_id(1)

    @pl.when(p == 0)
    def _stats_phase():
        z1 = jnp.concatenate([z1p_ref[0], z1p_ref[1]], axis=1)
        dinv = dinv_ref[...][:, 0:1]
        y1 = dinv * z1 + dinv * dinv * h_ref[...]
        y1_vmem[pl.ds(i * _BLK, _BLK), :] = y1
        s1 = jnp.sum(y1, axis=0, keepdims=True)
        s2 = jnp.sum(y1 * y1, axis=0, keepdims=True)

        @pl.when(i == 0)
        def _init():
            s1_ref[...] = s1
            s2_ref[...] = s2

        @pl.when(i != 0)
        def _acc():
            s1_ref[...] += s1
            s2_ref[...] += s2

    @pl.when(p == 1)
    def _apply_phase():
        mean = s1_ref[...] / _N
        var = s2_ref[...] / _N - mean * mean
        scale = lax.rsqrt(var + 1e-5) * gamma_ref[...]
        shift = beta_ref[...] - mean * scale
        y1 = y1_vmem[pl.ds(i * _BLK, _BLK), :]
        x2 = jnp.maximum(y1 * scale + shift, 0.0)
        y = jnp.dot(x2, w2_ref[...], preferred_element_type=jnp.float32)
        y_ref[...] = y
        g2_ref[...] = dinv_ref[...] * y


_bn_fused = pl.pallas_call(
    _bn_fused_body,
    grid=(2, _GRID),
    in_specs=[
        pl.BlockSpec((_NC, _BLK, 64), lambda p, i: (0, i * (1 - p), 0)),
        pl.BlockSpec((_BLK, _H), lambda p, i: (i * (1 - p), 0)),
        pl.BlockSpec((_BLK, 8), lambda p, i: (i, 0)),
        pl.BlockSpec((1, _H), lambda p, i: (0, 0)),
        pl.BlockSpec((1, _H), lambda p, i: (0, 0)),
        pl.BlockSpec((_H, 1), lambda p, i: (0, 0)),
    ],
    out_specs=[
        pl.BlockSpec((_BLK, 1), lambda p, i: (i, 0)),
        pl.BlockSpec((_BLK, 8), lambda p, i: (i, 0)),
    ],
    out_shape=[
        jax.ShapeDtypeStruct((_N, 1), jnp.float32),
        jax.ShapeDtypeStruct((_N, 8), jnp.float32),
    ],
    scratch_shapes=[
        pltpu.VMEM((_N, _H), jnp.float32),
        pltpu.VMEM((1, _H), jnp.float32),
        pltpu.VMEM((1, _H), jnp.float32),
    ],
)


def _out_body(z2p_ref, y_ref, dinv_ref, b2_ref, o_ref):
    dinv = dinv_ref[...][:, 0:1]
    z2 = (z2p_ref[0] + z2p_ref[1])[:, 0:1]
    o_ref[...] = dinv * z2 + dinv * dinv * y_ref[...] + b2_ref[0, 0]


_out_k = pl.pallas_call(
    _out_body,
    grid=(_GRID,),
    in_specs=[
        pl.BlockSpec((_NC, _BLK, 8), lambda i: (0, i, 0)),
        pl.BlockSpec((_BLK, 1), lambda i: (i, 0)),
        pl.BlockSpec((_BLK, 8), lambda i: (i, 0)),
        pl.BlockSpec((1, 1), lambda i: (0, 0)),
    ],
    out_specs=pl.BlockSpec((_BLK, 1), lambda i: (i, 0)),
    out_shape=jax.ShapeDtypeStruct((_N, 1), jnp.float32),
)


def kernel(features, edge_index, W1, b1, gamma, beta, W2, b2):
    del b1
    zeros64 = jnp.zeros((_N, 64), jnp.float32)
    zeros8 = jnp.zeros((_N, 8), jnp.float32)
    ones8 = jnp.ones((_KA, 8), jnp.float32)

    degp = _deg_kernel(edge_index, ones8, zeros8)
    h, g, dinv8 = _mm_scale(features, W1, degp)
    z1p = _agg64(edge_index, g, zeros64)
    y, g2 = _bn_fused(z1p, h, dinv8, gamma.reshape(1, _H),
                      beta.reshape(1, _H), W2)
    z2p = _agg8(edge_index, g2, zeros8)
    return _out_k(z2p, y, dinv8, b2.reshape(1, 1))


# speedup vs baseline: 1.0646x; 1.0646x over previous
import jax
import jax.numpy as jnp
from jax import lax
from jax.experimental import pallas as pl
from jax.experimental.pallas import tpu as pltpu
from jax.experimental.pallas import tpu_sc as plsc

_N = 10000
_E = 320000
_D = 128
_H = 128

_NC = 2
_NS = 16
_RA = 624
_RL = _N - 15 * _RA
_EPC = _E // _NC
_EPT = _EPC // _NS

_KA = 2000
_KB = 200
_NCHB = _E // _NS // _KB
_EPTB = _E // _NS
_KC = 1000
_NCHC = _EPT // _KC

_BLK = 1000
_GRID = _N // _BLK


def _sc_mesh():
    return plsc.VectorSubcoreMesh(core_axis_name="c", subcore_axis_name="s",
                                  num_cores=_NC, num_subcores=_NS)


def _tile_slab_copy(sid, copy_fn):
    @pl.when(sid < _NS - 1)
    def _body():
        copy_fn(sid * _RA, _RA)

    @pl.when(sid == _NS - 1)
    def _last():
        copy_fn((_NS - 1) * _RA, _RL)


def _deg_body(ei_h, ones_h, zero_h, out_h, acc, idx_d, ones_v):
    cid = lax.axis_index("c")
    sid = lax.axis_index("s")
    _tile_slab_copy(sid, lambda rb, nr: pltpu.sync_copy(
        zero_h.at[pl.ds(rb, nr), :], acc.at[pl.ds(rb, nr), :]))
    pltpu.sync_copy(ones_h, ones_v)
    plsc.subcore_barrier()
    ebase = cid * _EPC + sid * _EPT

    def chunk(i, carry):
        off = ebase + i * _KA
        pltpu.sync_copy(ei_h.at[1, pl.ds(off, _KA)], idx_d)
        pltpu.sync_copy(ones_v, acc.at[idx_d], add=True)
        return carry

    lax.fori_loop(0, _EPT // _KA, chunk, 0)
    plsc.subcore_barrier()
    _tile_slab_copy(sid, lambda rb, nr: pltpu.sync_copy(
        acc.at[pl.ds(rb, nr), :], out_h.at[cid, pl.ds(rb, nr), :]))


_SC_PARAMS = pltpu.CompilerParams(use_tc_tiling_on_sc=False)

_deg_kernel = pl.kernel(
    _deg_body,
    out_type=jax.ShapeDtypeStruct((_NC, _N, 8), jnp.float32),
    mesh=_sc_mesh(),
    compiler_params=_SC_PARAMS,
    scratch_types=[
        pltpu.VMEM_SHARED((_N, 8), jnp.float32),
        pltpu.VMEM((_KA,), jnp.int32),
        pltpu.VMEM((_KA, 8), jnp.float32),
    ],
)


def _pipelined_edge_loop(ei_h, k, nch, ebase, gather_view_fn, acc,
                         idx_s, idx_d, rows, isem, gsem, ssem):
    for j in range(2):
        pltpu.async_copy(ei_h.at[0, pl.ds(ebase + j * k, k)], idx_s.at[j],
                         isem)
        pltpu.async_copy(ei_h.at[1, pl.ds(ebase + j * k, k)], idx_d.at[j],
                         isem)

    def chunk(i, carry):
        b3 = lax.rem(i, 3)
        b4 = lax.rem(i, 4)
        off = ebase + i * k
        pltpu.make_async_copy(ei_h.at[0, pl.ds(off, k)], idx_s.at[b4],
                              isem).wait()
        pltpu.make_async_copy(ei_h.at[1, pl.ds(off, k)], idx_d.at[b4],
                              isem).wait()
        gd = pltpu.async_copy(gather_view_fn(idx_s.at[b4]), rows.at[b3],
                              gsem)

        @pl.when(i >= 2)
        def _wait_scatter_i_minus_2():
            pltpu.make_async_copy(rows.at[lax.rem(i - 2, 3)],
                                  acc.at[idx_d.at[lax.rem(i - 2, 4)]],
                                  ssem).wait()

        @pl.when(i + 2 < nch)
        def _prefetch_idx():
            nb4 = lax.rem(i + 2, 4)
            pltpu.async_copy(ei_h.at[0, pl.ds(off + 2 * k, k)],
                             idx_s.at[nb4], isem)
            pltpu.async_copy(ei_h.at[1, pl.ds(off + 2 * k, k)],
                             idx_d.at[nb4], isem)

        gd.wait()
        pltpu.async_copy(rows.at[b3], acc.at[idx_d.at[b4]], ssem, add=True)
        return carry

    lax.fori_loop(0, nch, chunk, 0)
    for j in (nch - 2, nch - 1):
        pltpu.make_async_copy(rows.at[j % 3], acc.at[idx_d.at[j % 4]],
                              ssem).wait()


def _agg64_body(ei_h, tab_h, zero_h, out_h, acc, tab_sp, idx_s, idx_d,
                rows, isem, gsem, ssem):
    cid = lax.axis_index("c")
    sid = lax.axis_index("s")

    def _init(rb, nr):
        pltpu.sync_copy(zero_h.at[pl.ds(rb, nr), :], acc.at[pl.ds(rb, nr), :])
        pltpu.sync_copy(tab_h.at[cid, pl.ds(rb, nr), :],
                        tab_sp.at[pl.ds(rb, nr), :])

    _tile_slab_copy(sid, _init)
    plsc.subcore_barrier()
    _pipelined_edge_loop(ei_h, _KB, _NCHB, sid * _EPTB,
                         lambda idx: tab_sp.at[idx], acc,
                         idx_s, idx_d, rows, isem, gsem, ssem)
    plsc.subcore_barrier()
    _tile_slab_copy(sid, lambda rb, nr: pltpu.sync_copy(
        acc.at[pl.ds(rb, nr), :], out_h.at[cid, pl.ds(rb, nr), :]))


_agg64 = pl.kernel(
    _agg64_body,
    out_type=jax.ShapeDtypeStruct((_NC, _N, 64), jnp.float32),
    mesh=_sc_mesh(),
    compiler_params=_SC_PARAMS,
    scratch_types=[
        pltpu.VMEM_SHARED((_N, 64), jnp.float32),
        pltpu.VMEM_SHARED((_N, 64), jnp.float32),
        pltpu.VMEM((4, _KB), jnp.int32),
        pltpu.VMEM((4, _KB), jnp.int32),
        pltpu.VMEM((3, _KB, 64), jnp.float32),
        pltpu.SemaphoreType.DMA,
        pltpu.SemaphoreType.DMA,
        pltpu.SemaphoreType.DMA,
    ],
)


def _agg8_body(ei_h, tab_h, zero_h, out_h, acc, tab_sp, idx_s, idx_d,
               rows, isem, gsem, ssem):
    cid = lax.axis_index("c")
    sid = lax.axis_index("s")

    def _init(rb, nr):
        pltpu.sync_copy(zero_h.at[pl.ds(rb, nr), :], acc.at[pl.ds(rb, nr), :])
        pltpu.sync_copy(tab_h.at[pl.ds(rb, nr), :], tab_sp.at[pl.ds(rb, nr), :])

    _tile_slab_copy(sid, _init)
    plsc.subcore_barrier()
    _pipelined_edge_loop(ei_h, _KC, _NCHC, cid * _EPC + sid * _EPT,
                         lambda idx: tab_sp.at[idx], acc,
                         idx_s, idx_d, rows, isem, gsem, ssem)
    plsc.subcore_barrier()
    _tile_slab_copy(sid, lambda rb, nr: pltpu.sync_copy(
        acc.at[pl.ds(rb, nr), :], out_h.at[cid, pl.ds(rb, nr), :]))


_agg8 = pl.kernel(
    _agg8_body,
    out_type=jax.ShapeDtypeStruct((_NC, _N, 8), jnp.float32),
    mesh=_sc_mesh(),
    compiler_params=_SC_PARAMS,
    scratch_types=[
        pltpu.VMEM_SHARED((_N, 8), jnp.float32),
        pltpu.VMEM_SHARED((_N, 8), jnp.float32),
        pltpu.VMEM((4, _KC), jnp.int32),
        pltpu.VMEM((4, _KC), jnp.int32),
        pltpu.VMEM((3, _KC, 8), jnp.float32),
        pltpu.SemaphoreType.DMA,
        pltpu.SemaphoreType.DMA,
        pltpu.SemaphoreType.DMA,
    ],
)


def _mm_scale_body(x_ref, w_ref, degp_ref, h_ref, g_ref, dinv_ref):
    h = jnp.dot(x_ref[...], w_ref[...], preferred_element_type=jnp.float32)
    deg = degp_ref[0] + degp_ref[1] + 1.0
    dinv = lax.rsqrt(deg)
    h_ref[...] = h
    g = h * dinv[:, 0:1]
    g_ref[...] = jnp.stack([g[:, :64], g[:, 64:]])
    dinv_ref[...] = dinv


_mm_scale = pl.pallas_call(
    _mm_scale_body,
    grid=(_GRID,),
    in_specs=[
        pl.BlockSpec((_BLK, _D), lambda i: (i, 0)),
        pl.BlockSpec((_D, _H), lambda i: (0, 0)),
        pl.BlockSpec((_NC, _BLK, 8), lambda i: (0, i, 0)),
    ],
    out_specs=[
        pl.BlockSpec((_BLK, _H), lambda i: (i, 0)),
        pl.BlockSpec((_NC, _BLK, 64), lambda i: (0, i, 0)),
        pl.BlockSpec((_BLK, 8), lambda i: (i, 0)),
    ],
    out_shape=[
        jax.ShapeDtypeStruct((_N, _H), jnp.float32),
        jax.ShapeDtypeStruct((_NC, _N, 64), jnp.float32),
        jax.ShapeDtypeStruct((_N, 8), jnp.float32),
    ],
)


def _bn_fused_body(z1p_ref, h_ref, dinv_ref, gamma_ref, beta_ref, w2_ref,
                   y_ref, g2_ref, y1_vmem, s1_ref, s2_ref):
    p = pl.program_id(0)
    i = pl.program_id(1)

    @pl.when(p == 0)
    def _stats_phase():
        z1 = jnp.concatenate([z1p_ref[0], z1p_ref[1]], axis=1)
        dinv = dinv_ref[...][:, 0:1]
        y1 = dinv * z1 + dinv * dinv * h_ref[...]
        y1_vmem[pl.ds(i * _BLK, _BLK), :] = y1
        s1 = jnp.sum(y1, axis=0, keepdims=True)
        s2 = jnp.sum(y1 * y1, axis=0, keepdims=True)

        @pl.when(i == 0)
        def _init():
            s1_ref[...] = s1
            s2_ref[...] = s2

        @pl.when(i != 0)
        def _acc():
            s1_ref[...] += s1
            s2_ref[...] += s2

    @pl.when(p == 1)
    def _apply_phase():
        mean = s1_ref[...] / _N
        var = s2_ref[...] / _N - mean * mean
        scale = lax.rsqrt(var + 1e-5) * gamma_ref[...]
        shift = beta_ref[...] - mean * scale
        y1 = y1_vmem[pl.ds(i * _BLK, _BLK), :]
        x2 = jnp.maximum(y1 * scale + shift, 0.0)
        y = jnp.dot(x2, w2_ref[...], preferred_element_type=jnp.float32)
        y_ref[...] = y
        g2_ref[...] = dinv_ref[...] * y


_bn_fused = pl.pallas_call(
    _bn_fused_body,
    grid=(2, _GRID),
    in_specs=[
        pl.BlockSpec((_NC, _BLK, 64), lambda p, i: (0, i * (1 - p), 0)),
        pl.BlockSpec((_BLK, _H), lambda p, i: (i * (1 - p), 0)),
        pl.BlockSpec((_BLK, 8), lambda p, i: (i, 0)),
        pl.BlockSpec((1, _H), lambda p, i: (0, 0)),
        pl.BlockSpec((1, _H), lambda p, i: (0, 0)),
        pl.BlockSpec((_H, 1), lambda p, i: (0, 0)),
    ],
    out_specs=[
        pl.BlockSpec((_BLK, 1), lambda p, i: (i, 0)),
        pl.BlockSpec((_BLK, 8), lambda p, i: (i, 0)),
    ],
    out_shape=[
        jax.ShapeDtypeStruct((_N, 1), jnp.float32),
        jax.ShapeDtypeStruct((_N, 8), jnp.float32),
    ],
    scratch_shapes=[
        pltpu.VMEM((_N, _H), jnp.float32),
        pltpu.VMEM((1, _H), jnp.float32),
        pltpu.VMEM((1, _H), jnp.float32),
    ],
)


def _out_body(z2p_ref, y_ref, dinv_ref, b2_ref, o_ref):
    dinv = dinv_ref[...][:, 0:1]
    z2 = (z2p_ref[0] + z2p_ref[1])[:, 0:1]
    o_ref[...] = dinv * z2 + dinv * dinv * y_ref[...] + b2_ref[0, 0]


_out_k = pl.pallas_call(
    _out_body,
    grid=(_GRID,),
    in_specs=[
        pl.BlockSpec((_NC, _BLK, 8), lambda i: (0, i, 0)),
        pl.BlockSpec((_BLK, 1), lambda i: (i, 0)),
        pl.BlockSpec((_BLK, 8), lambda i: (i, 0)),
        pl.BlockSpec((1, 1), lambda i: (0, 0)),
    ],
    out_specs=pl.BlockSpec((_BLK, 1), lambda i: (i, 0)),
    out_shape=jax.ShapeDtypeStruct((_N, 1), jnp.float32),
)


def kernel(features, edge_index, W1, b1, gamma, beta, W2, b2):
    del b1
    zeros64 = jnp.zeros((_N, 64), jnp.float32)
    zeros8 = jnp.zeros((_N, 8), jnp.float32)
    ones8 = jnp.ones((_KA, 8), jnp.float32)

    degp = _deg_kernel(edge_index, ones8, zeros8)
    h, g, dinv8 = _mm_scale(features, W1, degp)
    z1p = _agg64(edge_index, g, zeros64)
    y, g2 = _bn_fused(z1p, h, dinv8, gamma.reshape(1, _H),
                      beta.reshape(1, _H), W2)
    z2p = _agg8(edge_index, g2, zeros8)
    return _out_k(z2p, y, dinv8, b2.reshape(1, 1))


# speedup vs baseline: 1.1522x; 1.0823x over previous
import jax
import jax.numpy as jnp
from jax import lax
from jax.experimental import pallas as pl
from jax.experimental.pallas import tpu as pltpu
from jax.experimental.pallas import tpu_sc as plsc

_N = 10000
_E = 320000
_D = 128
_H = 128

_NC = 2
_NS = 16
_RA = 624
_RL = _N - 15 * _RA
_EPC = _E // _NC
_EPT = _EPC // _NS

_KA = 2000
_KB = 200
_NCHB = _E // _NS // _KB
_EPTB = _E // _NS
_KC = 1000
_NCHC = _EPT // _KC

_BLK = 1000
_GRID = _N // _BLK


def _sc_mesh():
    return plsc.VectorSubcoreMesh(core_axis_name="c", subcore_axis_name="s",
                                  num_cores=_NC, num_subcores=_NS)


def _tile_slab_copy(sid, copy_fn):
    @pl.when(sid < _NS - 1)
    def _body():
        copy_fn(sid * _RA, _RA)

    @pl.when(sid == _NS - 1)
    def _last():
        copy_fn((_NS - 1) * _RA, _RL)


def _deg_body(ei_h, ones_h, zero_h, out_h, acc, idx_d, ones_v):
    cid = lax.axis_index("c")
    sid = lax.axis_index("s")
    _tile_slab_copy(sid, lambda rb, nr: pltpu.sync_copy(
        zero_h.at[pl.ds(rb, nr), :], acc.at[pl.ds(rb, nr), :]))
    pltpu.sync_copy(ones_h, ones_v)
    plsc.subcore_barrier()
    ebase = cid * _EPC + sid * _EPT

    def chunk(i, carry):
        off = ebase + i * _KA
        pltpu.sync_copy(ei_h.at[1, pl.ds(off, _KA)], idx_d)
        pltpu.sync_copy(ones_v, acc.at[idx_d], add=True)
        return carry

    lax.fori_loop(0, _EPT // _KA, chunk, 0)
    plsc.subcore_barrier()
    _tile_slab_copy(sid, lambda rb, nr: pltpu.sync_copy(
        acc.at[pl.ds(rb, nr), :], out_h.at[cid, pl.ds(rb, nr), :]))


_SC_PARAMS = pltpu.CompilerParams(use_tc_tiling_on_sc=False)

_deg_kernel = pl.kernel(
    _deg_body,
    out_type=jax.ShapeDtypeStruct((_NC, _N, 8), jnp.float32),
    mesh=_sc_mesh(),
    compiler_params=_SC_PARAMS,
    scratch_types=[
        pltpu.VMEM_SHARED((_N, 8), jnp.float32),
        pltpu.VMEM((_KA,), jnp.int32),
        pltpu.VMEM((_KA, 8), jnp.float32),
    ],
)


def _pipelined_edge_loop(ei_h, k, nch, ebase, gather_view_fn, acc,
                         idx_s, idx_d, rows, isem, gsem, ssem):
    for j in range(2):
        pltpu.async_copy(ei_h.at[0, pl.ds(ebase + j * k, k)], idx_s.at[j],
                         isem)
        pltpu.async_copy(ei_h.at[1, pl.ds(ebase + j * k, k)], idx_d.at[j],
                         isem)

    def chunk(i, carry):
        b3 = lax.rem(i, 3)
        b4 = lax.rem(i, 4)
        off = ebase + i * k
        pltpu.make_async_copy(ei_h.at[0, pl.ds(off, k)], idx_s.at[b4],
                              isem).wait()
        pltpu.make_async_copy(ei_h.at[1, pl.ds(off, k)], idx_d.at[b4],
                              isem).wait()
        gd = pltpu.async_copy(gather_view_fn(idx_s.at[b4]), rows.at[b3],
                              gsem)

        @pl.when(i >= 2)
        def _wait_scatter_i_minus_2():
            pltpu.make_async_copy(rows.at[lax.rem(i - 2, 3)],
                                  acc.at[idx_d.at[lax.rem(i - 2, 4)]],
                                  ssem).wait()

        @pl.when(i + 2 < nch)
        def _prefetch_idx():
            nb4 = lax.rem(i + 2, 4)
            pltpu.async_copy(ei_h.at[0, pl.ds(off + 2 * k, k)],
                             idx_s.at[nb4], isem)
            pltpu.async_copy(ei_h.at[1, pl.ds(off + 2 * k, k)],
                             idx_d.at[nb4], isem)

        gd.wait()
        pltpu.async_copy(rows.at[b3], acc.at[idx_d.at[b4]], ssem, add=True)
        return carry

    lax.fori_loop(0, nch, chunk, 0)
    for j in (nch - 2, nch - 1):
        pltpu.make_async_copy(rows.at[j % 3], acc.at[idx_d.at[j % 4]],
                              ssem).wait()


def _agg64_body(ei_h, tab_h, zero_h, out_h, acc, tab_sp, idx_s, idx_d,
                rows, isem, gsem, ssem):
    cid = lax.axis_index("c")
    sid = lax.axis_index("s")

    def _init(rb, nr):
        pltpu.sync_copy(zero_h.at[pl.ds(rb, nr), :], acc.at[pl.ds(rb, nr), :])
        pltpu.sync_copy(tab_h.at[pl.ds(rb, nr), pl.ds(cid * 64, 64)],
                        tab_sp.at[pl.ds(rb, nr), :])

    _tile_slab_copy(sid, _init)
    plsc.subcore_barrier()
    _pipelined_edge_loop(ei_h, _KB, _NCHB, sid * _EPTB,
                         lambda idx: tab_sp.at[idx], acc,
                         idx_s, idx_d, rows, isem, gsem, ssem)
    plsc.subcore_barrier()
    _tile_slab_copy(sid, lambda rb, nr: pltpu.sync_copy(
        acc.at[pl.ds(rb, nr), :],
        out_h.at[pl.ds(rb, nr), pl.ds(cid * 64, 64)]))


_agg64 = pl.kernel(
    _agg64_body,
    out_type=jax.ShapeDtypeStruct((_N, _H), jnp.float32),
    mesh=_sc_mesh(),
    compiler_params=_SC_PARAMS,
    scratch_types=[
        pltpu.VMEM_SHARED((_N, 64), jnp.float32),
        pltpu.VMEM_SHARED((_N, 64), jnp.float32),
        pltpu.VMEM((4, _KB), jnp.int32),
        pltpu.VMEM((4, _KB), jnp.int32),
        pltpu.VMEM((3, _KB, 64), jnp.float32),
        pltpu.SemaphoreType.DMA,
        pltpu.SemaphoreType.DMA,
        pltpu.SemaphoreType.DMA,
    ],
)


def _agg8_body(ei_h, tab_h, zero_h, out_h, acc, tab_sp, idx_s, idx_d,
               rows, isem, gsem, ssem):
    cid = lax.axis_index("c")
    sid = lax.axis_index("s")

    def _init(rb, nr):
        pltpu.sync_copy(zero_h.at[pl.ds(rb, nr), :], acc.at[pl.ds(rb, nr), :])
        pltpu.sync_copy(tab_h.at[pl.ds(rb, nr), :], tab_sp.at[pl.ds(rb, nr), :])

    _tile_slab_copy(sid, _init)
    plsc.subcore_barrier()
    _pipelined_edge_loop(ei_h, _KC, _NCHC, cid * _EPC + sid * _EPT,
                         lambda idx: tab_sp.at[idx], acc,
                         idx_s, idx_d, rows, isem, gsem, ssem)
    plsc.subcore_barrier()
    _tile_slab_copy(sid, lambda rb, nr: pltpu.sync_copy(
        acc.at[pl.ds(rb, nr), :], out_h.at[cid, pl.ds(rb, nr), :]))


_agg8 = pl.kernel(
    _agg8_body,
    out_type=jax.ShapeDtypeStruct((_NC, _N, 8), jnp.float32),
    mesh=_sc_mesh(),
    compiler_params=_SC_PARAMS,
    scratch_types=[
        pltpu.VMEM_SHARED((_N, 8), jnp.float32),
        pltpu.VMEM_SHARED((_N, 8), jnp.float32),
        pltpu.VMEM((4, _KC), jnp.int32),
        pltpu.VMEM((4, _KC), jnp.int32),
        pltpu.VMEM((3, _KC, 8), jnp.float32),
        pltpu.SemaphoreType.DMA,
        pltpu.SemaphoreType.DMA,
        pltpu.SemaphoreType.DMA,
    ],
)


def _mm_scale_body(x_ref, w_ref, degp_ref, h_ref, g_ref, dinv_ref):
    h = jnp.dot(x_ref[...], w_ref[...], preferred_element_type=jnp.float32)
    deg = degp_ref[0] + degp_ref[1] + 1.0
    dinv = lax.rsqrt(deg)
    h_ref[...] = h
    g_ref[...] = h * dinv[:, 0:1]
    dinv_ref[...] = dinv


_mm_scale = pl.pallas_call(
    _mm_scale_body,
    grid=(_GRID,),
    in_specs=[
        pl.BlockSpec((_BLK, _D), lambda i: (i, 0)),
        pl.BlockSpec((_D, _H), lambda i: (0, 0)),
        pl.BlockSpec((_NC, _BLK, 8), lambda i: (0, i, 0)),
    ],
    out_specs=[
        pl.BlockSpec((_BLK, _H), lambda i: (i, 0)),
        pl.BlockSpec((_BLK, _H), lambda i: (i, 0)),
        pl.BlockSpec((_BLK, 8), lambda i: (i, 0)),
    ],
    out_shape=[
        jax.ShapeDtypeStruct((_N, _H), jnp.float32),
        jax.ShapeDtypeStruct((_N, _H), jnp.float32),
        jax.ShapeDtypeStruct((_N, 8), jnp.float32),
    ],
)


def _bn_fused_body(z1p_ref, h_ref, dinv_ref, gamma_ref, beta_ref, w2_ref,
                   y_ref, g2_ref, y1_vmem, s1_ref, s2_ref):
    p = pl.program_id(0)
    i = pl.program_id(1)

    @pl.when(p == 0)
    def _stats_phase():
        dinv = dinv_ref[...][:, 0:1]
        y1 = dinv * z1p_ref[...] + dinv * dinv * h_ref[...]
        y1_vmem[pl.ds(i * _BLK, _BLK), :] = y1
        s1 = jnp.sum(y1, axis=0, keepdims=True)
        s2 = jnp.sum(y1 * y1, axis=0, keepdims=True)

        @pl.when(i == 0)
        def _init():
            s1_ref[...] = s1
            s2_ref[...] = s2

        @pl.when(i != 0)
        def _acc():
            s1_ref[...] += s1
            s2_ref[...] += s2

    @pl.when(p == 1)
    def _apply_phase():
        mean = s1_ref[...] / _N
        var = s2_ref[...] / _N - mean * mean
        scale = lax.rsqrt(var + 1e-5) * gamma_ref[...]
        shift = beta_ref[...] - mean * scale
        y1 = y1_vmem[pl.ds(i * _BLK, _BLK), :]
        x2 = jnp.maximum(y1 * scale + shift, 0.0)
        y = jnp.dot(x2, w2_ref[...], preferred_element_type=jnp.float32)
        y_ref[...] = y
        g2_ref[...] = dinv_ref[...] * y


_bn_fused = pl.pallas_call(
    _bn_fused_body,
    grid=(2, _GRID),
    in_specs=[
        pl.BlockSpec((_BLK, _H), lambda p, i: (i * (1 - p), 0)),
        pl.BlockSpec((_BLK, _H), lambda p, i: (i * (1 - p), 0)),
        pl.BlockSpec((_BLK, 8), lambda p, i: (i, 0)),
        pl.BlockSpec((1, _H), lambda p, i: (0, 0)),
        pl.BlockSpec((1, _H), lambda p, i: (0, 0)),
        pl.BlockSpec((_H, 1), lambda p, i: (0, 0)),
    ],
    out_specs=[
        pl.BlockSpec((_BLK, 1), lambda p, i: (i, 0)),
        pl.BlockSpec((_BLK, 8), lambda p, i: (i, 0)),
    ],
    out_shape=[
        jax.ShapeDtypeStruct((_N, 1), jnp.float32),
        jax.ShapeDtypeStruct((_N, 8), jnp.float32),
    ],
    scratch_shapes=[
        pltpu.VMEM((_N, _H), jnp.float32),
        pltpu.VMEM((1, _H), jnp.float32),
        pltpu.VMEM((1, _H), jnp.float32),
    ],
)


def _out_body(z2p_ref, y_ref, dinv_ref, b2_ref, o_ref):
    dinv = dinv_ref[...][:, 0:1]
    z2 = (z2p_ref[0] + z2p_ref[1])[:, 0:1]
    o_ref[...] = dinv * z2 + dinv * dinv * y_ref[...] + b2_ref[0, 0]


_out_k = pl.pallas_call(
    _out_body,
    grid=(_GRID,),
    in_specs=[
        pl.BlockSpec((_NC, _BLK, 8), lambda i: (0, i, 0)),
        pl.BlockSpec((_BLK, 1), lambda i: (i, 0)),
        pl.BlockSpec((_BLK, 8), lambda i: (i, 0)),
        pl.BlockSpec((1, 1), lambda i: (0, 0)),
    ],
    out_specs=pl.BlockSpec((_BLK, 1), lambda i: (i, 0)),
    out_shape=jax.ShapeDtypeStruct((_N, 1), jnp.float32),
)


def kernel(features, edge_index, W1, b1, gamma, beta, W2, b2):
    del b1
    zeros64 = jnp.zeros((_N, 64), jnp.float32)
    zeros8 = jnp.zeros((_N, 8), jnp.float32)
    ones8 = jnp.ones((_KA, 8), jnp.float32)

    degp = _deg_kernel(edge_index, ones8, zeros8)
    h, g, dinv8 = _mm_scale(features, W1, degp)
    z1 = _agg64(edge_index, g, zeros64)
    y, g2 = _bn_fused(z1, h, dinv8, gamma.reshape(1, _H),
                      beta.reshape(1, _H), W2)
    z2p = _agg8(edge_index, g2, zeros8)
    return _out_k(z2p, y, dinv8, b2.reshape(1, 1))


# speedup vs baseline: 1.1594x; 1.0062x over previous
import jax
import jax.numpy as jnp
from jax import lax
from jax.experimental import pallas as pl
from jax.experimental.pallas import tpu as pltpu
from jax.experimental.pallas import tpu_sc as plsc

_N = 10000
_E = 320000
_D = 128
_H = 128

_NC = 2
_NS = 16
_RA = 624
_RL = _N - 15 * _RA
_EPC = _E // _NC
_EPT = _EPC // _NS

_KA = 2000
_KB = 200
_NCHB = _E // _NS // _KB
_EPTB = _E // _NS
_KC = 1000
_NCHC = _EPT // _KC

_BLK = 1000
_GRID = _N // _BLK


def _sc_mesh():
    return plsc.VectorSubcoreMesh(core_axis_name="c", subcore_axis_name="s",
                                  num_cores=_NC, num_subcores=_NS)


def _tile_slab_copy(sid, copy_fn):
    @pl.when(sid < _NS - 1)
    def _body():
        copy_fn(sid * _RA, _RA)

    @pl.when(sid == _NS - 1)
    def _last():
        copy_fn((_NS - 1) * _RA, _RL)


def _deg_body(ei_h, ones_h, zero_h, out_h, acc, idx_d, ones_v):
    cid = lax.axis_index("c")
    sid = lax.axis_index("s")
    _tile_slab_copy(sid, lambda rb, nr: pltpu.sync_copy(
        zero_h.at[pl.ds(rb, nr), :], acc.at[pl.ds(rb, nr), :]))
    pltpu.sync_copy(ones_h, ones_v)
    plsc.subcore_barrier()
    ebase = cid * _EPC + sid * _EPT

    def chunk(i, carry):
        off = ebase + i * _KA
        pltpu.sync_copy(ei_h.at[1, pl.ds(off, _KA)], idx_d)
        pltpu.sync_copy(ones_v, acc.at[idx_d], add=True)
        return carry

    lax.fori_loop(0, _EPT // _KA, chunk, 0)
    plsc.subcore_barrier()
    _tile_slab_copy(sid, lambda rb, nr: pltpu.sync_copy(
        acc.at[pl.ds(rb, nr), :],
        out_h.at[pl.ds(rb, nr), pl.ds(cid * 16, 16)]))


_SC_PARAMS = pltpu.CompilerParams(use_tc_tiling_on_sc=False)

_deg_kernel = pl.kernel(
    _deg_body,
    out_type=jax.ShapeDtypeStruct((_N, _H), jnp.float32),
    mesh=_sc_mesh(),
    compiler_params=_SC_PARAMS,
    scratch_types=[
        pltpu.VMEM_SHARED((_N, 16), jnp.float32),
        pltpu.VMEM((_KA,), jnp.int32),
        pltpu.VMEM((_KA, 16), jnp.float32),
    ],
)


def _pipelined_edge_loop(ei_h, k, nch, ebase, gather_view_fn, acc,
                         idx_s, idx_d, rows, isem, gsem, ssem):
    for j in range(2):
        pltpu.async_copy(ei_h.at[0, pl.ds(ebase + j * k, k)], idx_s.at[j],
                         isem)
        pltpu.async_copy(ei_h.at[1, pl.ds(ebase + j * k, k)], idx_d.at[j],
                         isem)

    def chunk(i, carry):
        b3 = lax.rem(i, 3)
        b4 = lax.rem(i, 4)
        off = ebase + i * k
        pltpu.make_async_copy(ei_h.at[0, pl.ds(off, k)], idx_s.at[b4],
                              isem).wait()
        pltpu.make_async_copy(ei_h.at[1, pl.ds(off, k)], idx_d.at[b4],
                              isem).wait()
        gd = pltpu.async_copy(gather_view_fn(idx_s.at[b4]), rows.at[b3],
                              gsem)

        @pl.when(i >= 2)
        def _wait_scatter_i_minus_2():
            pltpu.make_async_copy(rows.at[lax.rem(i - 2, 3)],
                                  acc.at[idx_d.at[lax.rem(i - 2, 4)]],
                                  ssem).wait()

        @pl.when(i + 2 < nch)
        def _prefetch_idx():
            nb4 = lax.rem(i + 2, 4)
            pltpu.async_copy(ei_h.at[0, pl.ds(off + 2 * k, k)],
                             idx_s.at[nb4], isem)
            pltpu.async_copy(ei_h.at[1, pl.ds(off + 2 * k, k)],
                             idx_d.at[nb4], isem)

        gd.wait()
        pltpu.async_copy(rows.at[b3], acc.at[idx_d.at[b4]], ssem, add=True)
        return carry

    lax.fori_loop(0, nch, chunk, 0)
    for j in (nch - 2, nch - 1):
        pltpu.make_async_copy(rows.at[j % 3], acc.at[idx_d.at[j % 4]],
                              ssem).wait()


def _agg64_body(ei_h, tab_h, zero_h, out_h, acc, tab_sp, idx_s, idx_d,
                rows, isem, gsem, ssem):
    cid = lax.axis_index("c")
    sid = lax.axis_index("s")

    def _init(rb, nr):
        pltpu.sync_copy(zero_h.at[pl.ds(rb, nr), :], acc.at[pl.ds(rb, nr), :])
        pltpu.sync_copy(tab_h.at[pl.ds(rb, nr), pl.ds(cid * 64, 64)],
                        tab_sp.at[pl.ds(rb, nr), :])

    _tile_slab_copy(sid, _init)
    plsc.subcore_barrier()
    _pipelined_edge_loop(ei_h, _KB, _NCHB, sid * _EPTB,
                         lambda idx: tab_sp.at[idx], acc,
                         idx_s, idx_d, rows, isem, gsem, ssem)
    plsc.subcore_barrier()
    _tile_slab_copy(sid, lambda rb, nr: pltpu.sync_copy(
        acc.at[pl.ds(rb, nr), :],
        out_h.at[pl.ds(rb, nr), pl.ds(cid * 64, 64)]))


_agg64 = pl.kernel(
    _agg64_body,
    out_type=jax.ShapeDtypeStruct((_N, _H), jnp.float32),
    mesh=_sc_mesh(),
    compiler_params=_SC_PARAMS,
    scratch_types=[
        pltpu.VMEM_SHARED((_N, 64), jnp.float32),
        pltpu.VMEM_SHARED((_N, 64), jnp.float32),
        pltpu.VMEM((4, _KB), jnp.int32),
        pltpu.VMEM((4, _KB), jnp.int32),
        pltpu.VMEM((3, _KB, 64), jnp.float32),
        pltpu.SemaphoreType.DMA,
        pltpu.SemaphoreType.DMA,
        pltpu.SemaphoreType.DMA,
    ],
)


def _agg8_body(ei_h, tab_h, zero_h, out_h, acc, tab_sp, idx_s, idx_d,
               rows, isem, gsem, ssem):
    cid = lax.axis_index("c")
    sid = lax.axis_index("s")

    def _init(rb, nr):
        pltpu.sync_copy(zero_h.at[pl.ds(rb, nr), :], acc.at[pl.ds(rb, nr), :])
        pltpu.sync_copy(tab_h.at[pl.ds(rb, nr), pl.ds(0, 16)],
                        tab_sp.at[pl.ds(rb, nr), :])

    _tile_slab_copy(sid, _init)
    plsc.subcore_barrier()
    _pipelined_edge_loop(ei_h, _KC, _NCHC, cid * _EPC + sid * _EPT,
                         lambda idx: tab_sp.at[idx], acc,
                         idx_s, idx_d, rows, isem, gsem, ssem)
    plsc.subcore_barrier()
    _tile_slab_copy(sid, lambda rb, nr: pltpu.sync_copy(
        acc.at[pl.ds(rb, nr), :],
        out_h.at[pl.ds(rb, nr), pl.ds(cid * 16, 16)]))


_agg8 = pl.kernel(
    _agg8_body,
    out_type=jax.ShapeDtypeStruct((_N, _H), jnp.float32),
    mesh=_sc_mesh(),
    compiler_params=_SC_PARAMS,
    scratch_types=[
        pltpu.VMEM_SHARED((_N, 16), jnp.float32),
        pltpu.VMEM_SHARED((_N, 16), jnp.float32),
        pltpu.VMEM((4, _KC), jnp.int32),
        pltpu.VMEM((4, _KC), jnp.int32),
        pltpu.VMEM((3, _KC, 16), jnp.float32),
        pltpu.SemaphoreType.DMA,
        pltpu.SemaphoreType.DMA,
        pltpu.SemaphoreType.DMA,
    ],
)


def _mm_scale_body(x_ref, w_ref, degp_ref, h_ref, g_ref, dinv_ref):
    h = jnp.dot(x_ref[...], w_ref[...], preferred_element_type=jnp.float32)
    degp = degp_ref[...]
    deg = degp[:, 0:1] + degp[:, 16:17] + 1.0
    dinv = lax.rsqrt(deg)
    h_ref[...] = h
    g_ref[...] = h * dinv
    dinv_ref[...] = jnp.broadcast_to(dinv, (_BLK, 8))


_mm_scale = pl.pallas_call(
    _mm_scale_body,
    grid=(_GRID,),
    in_specs=[
        pl.BlockSpec((_BLK, _D), lambda i: (i, 0)),
        pl.BlockSpec((_D, _H), lambda i: (0, 0)),
        pl.BlockSpec((_BLK, _H), lambda i: (i, 0)),
    ],
    out_specs=[
        pl.BlockSpec((_BLK, _H), lambda i: (i, 0)),
        pl.BlockSpec((_BLK, _H), lambda i: (i, 0)),
        pl.BlockSpec((_BLK, 8), lambda i: (i, 0)),
    ],
    out_shape=[
        jax.ShapeDtypeStruct((_N, _H), jnp.float32),
        jax.ShapeDtypeStruct((_N, _H), jnp.float32),
        jax.ShapeDtypeStruct((_N, 8), jnp.float32),
    ],
)


def _bn_fused_body(z1p_ref, h_ref, dinv_ref, gamma_ref, beta_ref, w2_ref,
                   y_ref, g2_ref, y1_vmem, s1_ref, s2_ref):
    p = pl.program_id(0)
    i = pl.program_id(1)

    @pl.when(p == 0)
    def _stats_phase():
        dinv = dinv_ref[...][:, 0:1]
        y1 = dinv * z1p_ref[...] + dinv * dinv * h_ref[...]
        y1_vmem[pl.ds(i * _BLK, _BLK), :] = y1
        s1 = jnp.sum(y1, axis=0, keepdims=True)
        s2 = jnp.sum(y1 * y1, axis=0, keepdims=True)

        @pl.when(i == 0)
        def _init():
            s1_ref[...] = s1
            s2_ref[...] = s2

        @pl.when(i != 0)
        def _acc():
            s1_ref[...] += s1
            s2_ref[...] += s2

    @pl.when(p == 1)
    def _apply_phase():
        mean = s1_ref[...] / _N
        var = s2_ref[...] / _N - mean * mean
        scale = lax.rsqrt(var + 1e-5) * gamma_ref[...]
        shift = beta_ref[...] - mean * scale
        y1 = y1_vmem[pl.ds(i * _BLK, _BLK), :]
        x2 = jnp.maximum(y1 * scale + shift, 0.0)
        y = jnp.dot(x2, w2_ref[...], preferred_element_type=jnp.float32)
        y_ref[...] = y
        g2_ref[...] = jnp.broadcast_to(dinv_ref[...][:, 0:1] * y,
                                       (_BLK, _H))


_bn_fused = pl.pallas_call(
    _bn_fused_body,
    grid=(2, _GRID),
    in_specs=[
        pl.BlockSpec((_BLK, _H), lambda p, i: (i * (1 - p), 0)),
        pl.BlockSpec((_BLK, _H), lambda p, i: (i * (1 - p), 0)),
        pl.BlockSpec((_BLK, 8), lambda p, i: (i, 0)),
        pl.BlockSpec((1, _H), lambda p, i: (0, 0)),
        pl.BlockSpec((1, _H), lambda p, i: (0, 0)),
        pl.BlockSpec((_H, 1), lambda p, i: (0, 0)),
    ],
    out_specs=[
        pl.BlockSpec((_BLK, 1), lambda p, i: (i, 0)),
        pl.BlockSpec((_BLK, _H), lambda p, i: (i, 0)),
    ],
    out_shape=[
        jax.ShapeDtypeStruct((_N, 1), jnp.float32),
        jax.ShapeDtypeStruct((_N, _H), jnp.float32),
    ],
    scratch_shapes=[
        pltpu.VMEM((_N, _H), jnp.float32),
        pltpu.VMEM((1, _H), jnp.float32),
        pltpu.VMEM((1, _H), jnp.float32),
    ],
)


def _out_body(z2p_ref, y_ref, dinv_ref, b2_ref, o_ref):
    dinv = dinv_ref[...][:, 0:1]
    z2p = z2p_ref[...]
    z2 = z2p[:, 0:1] + z2p[:, 16:17]
    o_ref[...] = dinv * z2 + dinv * dinv * y_ref[...] + b2_ref[0, 0]


_out_k = pl.pallas_call(
    _out_body,
    grid=(_GRID,),
    in_specs=[
        pl.BlockSpec((_BLK, _H), lambda i: (i, 0)),
        pl.BlockSpec((_BLK, 1), lambda i: (i, 0)),
        pl.BlockSpec((_BLK, 8), lambda i: (i, 0)),
        pl.BlockSpec((1, 1), lambda i: (0, 0)),
    ],
    out_specs=pl.BlockSpec((_BLK, 1), lambda i: (i, 0)),
    out_shape=jax.ShapeDtypeStruct((_N, 1), jnp.float32),
)


def kernel(features, edge_index, W1, b1, gamma, beta, W2, b2):
    del b1
    zeros64 = jnp.zeros((_N, 64), jnp.float32)
    zeros16 = jnp.zeros((_N, 16), jnp.float32)
    ones16 = jnp.ones((_KA, 16), jnp.float32)

    degp = _deg_kernel(edge_index, ones16, zeros16)
    h, g, dinv8 = _mm_scale(features, W1, degp)
    z1 = _agg64(edge_index, g, zeros64)
    y, g2 = _bn_fused(z1, h, dinv8, gamma.reshape(1, _H),
                      beta.reshape(1, _H), W2)
    z2p = _agg8(edge_index, g2, zeros16)
    return _out_k(z2p, y, dinv8, b2.reshape(1, 1))


# speedup vs baseline: 1.2169x; 1.0496x over previous
import jax
import jax.numpy as jnp
from jax import lax
from jax.experimental import pallas as pl
from jax.experimental.pallas import tpu as pltpu
from jax.experimental.pallas import tpu_sc as plsc

_N = 10000
_E = 320000
_D = 128
_H = 128

_NC = 2
_NS = 16
_RA = 624
_RL = _N - 15 * _RA
_EPC = _E // _NC
_EPT = _EPC // _NS

_KA = 2000
_KB = 200
_NCHB = _E // _NS // _KB
_EPTB = _E // _NS
_KC = 1000
_NCHC = _EPT // _KC

_BLK = 1000
_GRID = _N // _BLK


def _sc_mesh():
    return plsc.VectorSubcoreMesh(core_axis_name="c", subcore_axis_name="s",
                                  num_cores=_NC, num_subcores=_NS)


def _tile_slab_copy(sid, copy_fn):
    @pl.when(sid < _NS - 1)
    def _body():
        copy_fn(sid * _RA, _RA)

    @pl.when(sid == _NS - 1)
    def _last():
        copy_fn((_NS - 1) * _RA, _RL)


def _deg_body(ei_h, ones_h, zero_h, out_h, acc, idx_d, ones_v):
    cid = lax.axis_index("c")
    sid = lax.axis_index("s")
    _tile_slab_copy(sid, lambda rb, nr: pltpu.sync_copy(
        zero_h.at[pl.ds(rb, nr), :], acc.at[pl.ds(rb, nr), :]))
    pltpu.sync_copy(ones_h, ones_v)
    plsc.subcore_barrier()
    ebase = cid * _EPC + sid * _EPT

    def chunk(i, carry):
        off = ebase + i * _KA
        pltpu.sync_copy(ei_h.at[1, pl.ds(off, _KA)], idx_d)
        pltpu.sync_copy(ones_v, acc.at[idx_d], add=True)
        return carry

    lax.fori_loop(0, _EPT // _KA, chunk, 0)
    plsc.subcore_barrier()
    _tile_slab_copy(sid, lambda rb, nr: pltpu.sync_copy(
        acc.at[pl.ds(rb, nr), :],
        out_h.at[pl.ds(rb, nr), pl.ds(cid * 8, 8)]))


_SC_PARAMS = pltpu.CompilerParams(use_tc_tiling_on_sc=False)

_deg_kernel = pl.kernel(
    _deg_body,
    out_type=jax.ShapeDtypeStruct((_N, _H), jnp.float32),
    mesh=_sc_mesh(),
    compiler_params=_SC_PARAMS,
    scratch_types=[
        pltpu.VMEM_SHARED((_N, 8), jnp.float32),
        pltpu.VMEM((_KA,), jnp.int32),
        pltpu.VMEM((_KA, 8), jnp.float32),
    ],
)


def _pipelined_edge_loop(ei_h, k, nch, ebase, gather_view_fn, acc,
                         idx_s, idx_d, rows, isem, gsem, ssem):
    for j in range(2):
        pltpu.async_copy(ei_h.at[0, pl.ds(ebase + j * k, k)], idx_s.at[j],
                         isem)
        pltpu.async_copy(ei_h.at[1, pl.ds(ebase + j * k, k)], idx_d.at[j],
                         isem)

    def chunk(i, carry):
        b3 = lax.rem(i, 3)
        b4 = lax.rem(i, 4)
        off = ebase + i * k
        pltpu.make_async_copy(ei_h.at[0, pl.ds(off, k)], idx_s.at[b4],
                              isem).wait()
        pltpu.make_async_copy(ei_h.at[1, pl.ds(off, k)], idx_d.at[b4],
                              isem).wait()
        gd = pltpu.async_copy(gather_view_fn(idx_s.at[b4]), rows.at[b3],
                              gsem)

        @pl.when(i >= 2)
        def _wait_scatter_i_minus_2():
            pltpu.make_async_copy(rows.at[lax.rem(i - 2, 3)],
                                  acc.at[idx_d.at[lax.rem(i - 2, 4)]],
                                  ssem).wait()

        @pl.when(i + 2 < nch)
        def _prefetch_idx():
            nb4 = lax.rem(i + 2, 4)
            pltpu.async_copy(ei_h.at[0, pl.ds(off + 2 * k, k)],
                             idx_s.at[nb4], isem)
            pltpu.async_copy(ei_h.at[1, pl.ds(off + 2 * k, k)],
                             idx_d.at[nb4], isem)

        gd.wait()
        pltpu.async_copy(rows.at[b3], acc.at[idx_d.at[b4]], ssem, add=True)
        return carry

    lax.fori_loop(0, nch, chunk, 0)
    for j in (nch - 2, nch - 1):
        pltpu.make_async_copy(rows.at[j % 3], acc.at[idx_d.at[j % 4]],
                              ssem).wait()


def _agg64_body(ei_h, tab_h, zero_h, out_h, acc, tab_sp, idx_s, idx_d,
                rows, isem, gsem, ssem):
    cid = lax.axis_index("c")
    sid = lax.axis_index("s")

    def _init(rb, nr):
        pltpu.sync_copy(zero_h.at[pl.ds(rb, nr), :], acc.at[pl.ds(rb, nr), :])
        pltpu.sync_copy(tab_h.at[pl.ds(rb, nr), pl.ds(cid * 64, 64)],
                        tab_sp.at[pl.ds(rb, nr), :])

    _tile_slab_copy(sid, _init)
    plsc.subcore_barrier()
    _pipelined_edge_loop(ei_h, _KB, _NCHB, sid * _EPTB,
                         lambda idx: tab_sp.at[idx], acc,
                         idx_s, idx_d, rows, isem, gsem, ssem)
    plsc.subcore_barrier()
    _tile_slab_copy(sid, lambda rb, nr: pltpu.sync_copy(
        acc.at[pl.ds(rb, nr), :],
        out_h.at[pl.ds(rb, nr), pl.ds(cid * 64, 64)]))


_agg64 = pl.kernel(
    _agg64_body,
    out_type=jax.ShapeDtypeStruct((_N, _H), jnp.float32),
    mesh=_sc_mesh(),
    compiler_params=_SC_PARAMS,
    scratch_types=[
        pltpu.VMEM_SHARED((_N, 64), jnp.float32),
        pltpu.VMEM_SHARED((_N, 64), jnp.float32),
        pltpu.VMEM((4, _KB), jnp.int32),
        pltpu.VMEM((4, _KB), jnp.int32),
        pltpu.VMEM((3, _KB, 64), jnp.float32),
        pltpu.SemaphoreType.DMA,
        pltpu.SemaphoreType.DMA,
        pltpu.SemaphoreType.DMA,
    ],
)


def _agg8_body(ei_h, tab_h, zero_h, out_h, acc, tab_sp, idx_s, idx_d,
               rows, isem, gsem, ssem):
    cid = lax.axis_index("c")
    sid = lax.axis_index("s")

    def _init(rb, nr):
        pltpu.sync_copy(zero_h.at[pl.ds(rb, nr), :], acc.at[pl.ds(rb, nr), :])
        pltpu.sync_copy(tab_h.at[pl.ds(rb, nr), pl.ds(0, 8)],
                        tab_sp.at[pl.ds(rb, nr), :])

    _tile_slab_copy(sid, _init)
    plsc.subcore_barrier()
    _pipelined_edge_loop(ei_h, _KC, _NCHC, cid * _EPC + sid * _EPT,
                         lambda idx: tab_sp.at[idx], acc,
                         idx_s, idx_d, rows, isem, gsem, ssem)
    plsc.subcore_barrier()
    _tile_slab_copy(sid, lambda rb, nr: pltpu.sync_copy(
        acc.at[pl.ds(rb, nr), :],
        out_h.at[pl.ds(rb, nr), pl.ds(cid * 8, 8)]))


_agg8 = pl.kernel(
    _agg8_body,
    out_type=jax.ShapeDtypeStruct((_N, _H), jnp.float32),
    mesh=_sc_mesh(),
    compiler_params=_SC_PARAMS,
    scratch_types=[
        pltpu.VMEM_SHARED((_N, 8), jnp.float32),
        pltpu.VMEM_SHARED((_N, 8), jnp.float32),
        pltpu.VMEM((4, _KC), jnp.int32),
        pltpu.VMEM((4, _KC), jnp.int32),
        pltpu.VMEM((3, _KC, 8), jnp.float32),
        pltpu.SemaphoreType.DMA,
        pltpu.SemaphoreType.DMA,
        pltpu.SemaphoreType.DMA,
    ],
)


def _mm_scale_body(x_ref, w_ref, degp_ref, h_ref, g_ref, dinv_ref):
    h = jnp.dot(x_ref[...], w_ref[...], preferred_element_type=jnp.float32)
    degp = degp_ref[...]
    deg = degp[:, 0:1] + degp[:, 8:9] + 1.0
    dinv = lax.rsqrt(deg)
    h_ref[...] = h
    g_ref[...] = h * dinv
    dinv_ref[...] = jnp.broadcast_to(dinv, (_BLK, 8))


_mm_scale = pl.pallas_call(
    _mm_scale_body,
    grid=(_GRID,),
    in_specs=[
        pl.BlockSpec((_BLK, _D), lambda i: (i, 0)),
        pl.BlockSpec((_D, _H), lambda i: (0, 0)),
        pl.BlockSpec((_BLK, _H), lambda i: (i, 0)),
    ],
    out_specs=[
        pl.BlockSpec((_BLK, _H), lambda i: (i, 0)),
        pl.BlockSpec((_BLK, _H), lambda i: (i, 0)),
        pl.BlockSpec((_BLK, 8), lambda i: (i, 0)),
    ],
    out_shape=[
        jax.ShapeDtypeStruct((_N, _H), jnp.float32),
        jax.ShapeDtypeStruct((_N, _H), jnp.float32),
        jax.ShapeDtypeStruct((_N, 8), jnp.float32),
    ],
)


def _bn_fused_body(z1p_ref, h_ref, dinv_ref, gamma_ref, beta_ref, w2_ref,
                   y_ref, g2_ref, y1_vmem, s1_ref, s2_ref):
    p = pl.program_id(0)
    i = pl.program_id(1)

    @pl.when(p == 0)
    def _stats_phase():
        dinv = dinv_ref[...][:, 0:1]
        y1 = dinv * z1p_ref[...] + dinv * dinv * h_ref[...]
        y1_vmem[pl.ds(i * _BLK, _BLK), :] = y1
        s1 = jnp.sum(y1, axis=0, keepdims=True)
        s2 = jnp.sum(y1 * y1, axis=0, keepdims=True)

        @pl.when(i == 0)
        def _init():
            s1_ref[...] = s1
            s2_ref[...] = s2

        @pl.when(i != 0)
        def _acc():
            s1_ref[...] += s1
            s2_ref[...] += s2

    @pl.when(p == 1)
    def _apply_phase():
        mean = s1_ref[...] / _N
        var = s2_ref[...] / _N - mean * mean
        scale = lax.rsqrt(var + 1e-5) * gamma_ref[...]
        shift = beta_ref[...] - mean * scale
        y1 = y1_vmem[pl.ds(i * _BLK, _BLK), :]
        x2 = jnp.maximum(y1 * scale + shift, 0.0)
        y = jnp.dot(x2, w2_ref[...], preferred_element_type=jnp.float32)
        y_ref[...] = y
        g2_ref[...] = jnp.broadcast_to(dinv_ref[...][:, 0:1] * y,
                                       (_BLK, _H))


_bn_fused = pl.pallas_call(
    _bn_fused_body,
    grid=(2, _GRID),
    in_specs=[
        pl.BlockSpec((_BLK, _H), lambda p, i: (i * (1 - p), 0)),
        pl.BlockSpec((_BLK, _H), lambda p, i: (i * (1 - p), 0)),
        pl.BlockSpec((_BLK, 8), lambda p, i: (i, 0)),
        pl.BlockSpec((1, _H), lambda p, i: (0, 0)),
        pl.BlockSpec((1, _H), lambda p, i: (0, 0)),
        pl.BlockSpec((_H, 1), lambda p, i: (0, 0)),
    ],
    out_specs=[
        pl.BlockSpec((_BLK, 1), lambda p, i: (i, 0)),
        pl.BlockSpec((_BLK, _H), lambda p, i: (i, 0)),
    ],
    out_shape=[
        jax.ShapeDtypeStruct((_N, 1), jnp.float32),
        jax.ShapeDtypeStruct((_N, _H), jnp.float32),
    ],
    scratch_shapes=[
        pltpu.VMEM((_N, _H), jnp.float32),
        pltpu.VMEM((1, _H), jnp.float32),
        pltpu.VMEM((1, _H), jnp.float32),
    ],
)


def _out_body(z2p_ref, y_ref, dinv_ref, b2_ref, o_ref):
    dinv = dinv_ref[...][:, 0:1]
    z2p = z2p_ref[...]
    z2 = z2p[:, 0:1] + z2p[:, 8:9]
    o_ref[...] = dinv * z2 + dinv * dinv * y_ref[...] + b2_ref[0, 0]


_out_k = pl.pallas_call(
    _out_body,
    grid=(_GRID,),
    in_specs=[
        pl.BlockSpec((_BLK, _H), lambda i: (i, 0)),
        pl.BlockSpec((_BLK, 1), lambda i: (i, 0)),
        pl.BlockSpec((_BLK, 8), lambda i: (i, 0)),
        pl.BlockSpec((1, 1), lambda i: (0, 0)),
    ],
    out_specs=pl.BlockSpec((_BLK, 1), lambda i: (i, 0)),
    out_shape=jax.ShapeDtypeStruct((_N, 1), jnp.float32),
)


def kernel(features, edge_index, W1, b1, gamma, beta, W2, b2):
    del b1
    zeros64 = jnp.zeros((_N, 64), jnp.float32)
    zeros8 = jnp.zeros((_N, 8), jnp.float32)
    ones8 = jnp.ones((_KA, 8), jnp.float32)

    degp = _deg_kernel(edge_index, ones8, zeros8)
    h, g, dinv8 = _mm_scale(features, W1, degp)
    z1 = _agg64(edge_index, g, zeros64)
    y, g2 = _bn_fused(z1, h, dinv8, gamma.reshape(1, _H),
                      beta.reshape(1, _H), W2)
    z2p = _agg8(edge_index, g2, zeros8)
    return _out_k(z2p, y, dinv8, b2.reshape(1, 1))


# speedup vs baseline: 1.2443x; 1.0225x over previous
import jax
import jax.numpy as jnp
from jax import lax
from jax.experimental import pallas as pl
from jax.experimental.pallas import tpu as pltpu
from jax.experimental.pallas import tpu_sc as plsc

_N = 10000
_E = 320000
_D = 128
_H = 128

_NC = 2
_NS = 16
_RA = 624
_RL = _N - 15 * _RA
_EPC = _E // _NC
_EPT = _EPC // _NS

_KA = 2000
_KB = 200
_NCHB = _E // _NS // _KB
_EPTB = _E // _NS
_KC = 1000
_NCHC = _EPT // _KC

_BLK = 1000
_GRID = _N // _BLK


def _sc_mesh():
    return plsc.VectorSubcoreMesh(core_axis_name="c", subcore_axis_name="s",
                                  num_cores=_NC, num_subcores=_NS)


def _tile_slab_copy(sid, copy_fn):
    @pl.when(sid < _NS - 1)
    def _body():
        copy_fn(sid * _RA, _RA)

    @pl.when(sid == _NS - 1)
    def _last():
        copy_fn((_NS - 1) * _RA, _RL)


def _deg_body(ei_h, ones_h, zero_h, out_h, acc, idx_d, ones_v):
    cid = lax.axis_index("c")
    sid = lax.axis_index("s")
    _tile_slab_copy(sid, lambda rb, nr: pltpu.sync_copy(
        zero_h.at[pl.ds(rb, nr), :], acc.at[pl.ds(rb, nr), :]))
    pltpu.sync_copy(ones_h, ones_v)
    plsc.subcore_barrier()
    ebase = cid * _EPC + sid * _EPT

    def chunk(i, carry):
        off = ebase + i * _KA
        pltpu.sync_copy(ei_h.at[1, pl.ds(off, _KA)], idx_d)
        pltpu.sync_copy(ones_v, acc.at[idx_d], add=True)
        return carry

    lax.fori_loop(0, _EPT // _KA, chunk, 0)
    plsc.subcore_barrier()
    _tile_slab_copy(sid, lambda rb, nr: pltpu.sync_copy(
        acc.at[pl.ds(rb, nr), :],
        out_h.at[pl.ds(rb, nr), pl.ds(cid * 8, 8)]))


_SC_PARAMS = pltpu.CompilerParams(use_tc_tiling_on_sc=False)

_deg_kernel = pl.kernel(
    _deg_body,
    out_type=jax.ShapeDtypeStruct((_N, _H), jnp.float32),
    mesh=_sc_mesh(),
    compiler_params=_SC_PARAMS,
    scratch_types=[
        pltpu.VMEM_SHARED((_N, 8), jnp.float32),
        pltpu.VMEM((_KA,), jnp.int32),
        pltpu.VMEM((_KA, 8), jnp.float32),
    ],
)


def _pipelined_edge_loop(ei_h, k, nch, ebase, gather_view_fn, acc,
                         idx_s, idx_d, rows, isem, gsem, ssem):
    for j in range(2):
        pltpu.async_copy(ei_h.at[0, pl.ds(ebase + j * k, k)], idx_s.at[j],
                         isem)
        pltpu.async_copy(ei_h.at[1, pl.ds(ebase + j * k, k)], idx_d.at[j],
                         isem)

    def chunk(i, carry):
        b3 = lax.rem(i, 3)
        b4 = lax.rem(i, 4)
        off = ebase + i * k
        pltpu.make_async_copy(ei_h.at[0, pl.ds(off, k)], idx_s.at[b4],
                              isem).wait()
        pltpu.make_async_copy(ei_h.at[1, pl.ds(off, k)], idx_d.at[b4],
                              isem).wait()
        gd = pltpu.async_copy(gather_view_fn(idx_s.at[b4]), rows.at[b3],
                              gsem)

        @pl.when(i >= 2)
        def _wait_scatter_i_minus_2():
            pltpu.make_async_copy(rows.at[lax.rem(i - 2, 3)],
                                  acc.at[idx_d.at[lax.rem(i - 2, 4)]],
                                  ssem).wait()

        @pl.when(i + 2 < nch)
        def _prefetch_idx():
            nb4 = lax.rem(i + 2, 4)
            pltpu.async_copy(ei_h.at[0, pl.ds(off + 2 * k, k)],
                             idx_s.at[nb4], isem)
            pltpu.async_copy(ei_h.at[1, pl.ds(off + 2 * k, k)],
                             idx_d.at[nb4], isem)

        gd.wait()
        pltpu.async_copy(rows.at[b3], acc.at[idx_d.at[b4]], ssem, add=True)
        return carry

    lax.fori_loop(0, nch, chunk, 0)
    for j in (nch - 2, nch - 1):
        pltpu.make_async_copy(rows.at[j % 3], acc.at[idx_d.at[j % 4]],
                              ssem).wait()


def _agg64_body(ei_h, tab_h, out_h, acc, tab_sp, idx_s, idx_d,
                rows, isem, gsem, ssem):
    cid = lax.axis_index("c")
    sid = lax.axis_index("s")

    def _init(rb, nr):
        pltpu.sync_copy(tab_h.at[pl.ds(rb, nr), pl.ds(cid * 64, 64)],
                        tab_sp.at[pl.ds(rb, nr), :])
        pltpu.sync_copy(tab_h.at[pl.ds(rb, nr), pl.ds(cid * 64, 64)],
                        acc.at[pl.ds(rb, nr), :])

    _tile_slab_copy(sid, _init)
    plsc.subcore_barrier()
    _pipelined_edge_loop(ei_h, _KB, _NCHB, sid * _EPTB,
                         lambda idx: tab_sp.at[idx], acc,
                         idx_s, idx_d, rows, isem, gsem, ssem)
    plsc.subcore_barrier()
    _tile_slab_copy(sid, lambda rb, nr: pltpu.sync_copy(
        acc.at[pl.ds(rb, nr), :],
        out_h.at[pl.ds(rb, nr), pl.ds(cid * 64, 64)]))


_agg64 = pl.kernel(
    _agg64_body,
    out_type=jax.ShapeDtypeStruct((_N, _H), jnp.float32),
    mesh=_sc_mesh(),
    compiler_params=_SC_PARAMS,
    scratch_types=[
        pltpu.VMEM_SHARED((_N, 64), jnp.float32),
        pltpu.VMEM_SHARED((_N, 64), jnp.float32),
        pltpu.VMEM((4, _KB), jnp.int32),
        pltpu.VMEM((4, _KB), jnp.int32),
        pltpu.VMEM((3, _KB, 64), jnp.float32),
        pltpu.SemaphoreType.DMA,
        pltpu.SemaphoreType.DMA,
        pltpu.SemaphoreType.DMA,
    ],
)


def _agg8_body(ei_h, tab_h, zero_h, out_h, acc, tab_sp, idx_s, idx_d,
               rows, isem, gsem, ssem):
    cid = lax.axis_index("c")
    sid = lax.axis_index("s")

    def _init(rb, nr):
        pltpu.sync_copy(tab_h.at[pl.ds(rb, nr), pl.ds(0, 8)],
                        tab_sp.at[pl.ds(rb, nr), :])

        @pl.when(cid == 0)
        def _self_loop():
            pltpu.sync_copy(tab_h.at[pl.ds(rb, nr), pl.ds(0, 8)],
                            acc.at[pl.ds(rb, nr), :])

        @pl.when(cid != 0)
        def _zero():
            pltpu.sync_copy(zero_h.at[pl.ds(rb, nr), :],
                            acc.at[pl.ds(rb, nr), :])

    _tile_slab_copy(sid, _init)
    plsc.subcore_barrier()
    _pipelined_edge_loop(ei_h, _KC, _NCHC, cid * _EPC + sid * _EPT,
                         lambda idx: tab_sp.at[idx], acc,
                         idx_s, idx_d, rows, isem, gsem, ssem)
    plsc.subcore_barrier()
    _tile_slab_copy(sid, lambda rb, nr: pltpu.sync_copy(
        acc.at[pl.ds(rb, nr), :],
        out_h.at[pl.ds(rb, nr), pl.ds(cid * 8, 8)]))


_agg8 = pl.kernel(
    _agg8_body,
    out_type=jax.ShapeDtypeStruct((_N, _H), jnp.float32),
    mesh=_sc_mesh(),
    compiler_params=_SC_PARAMS,
    scratch_types=[
        pltpu.VMEM_SHARED((_N, 8), jnp.float32),
        pltpu.VMEM_SHARED((_N, 8), jnp.float32),
        pltpu.VMEM((4, _KC), jnp.int32),
        pltpu.VMEM((4, _KC), jnp.int32),
        pltpu.VMEM((3, _KC, 8), jnp.float32),
        pltpu.SemaphoreType.DMA,
        pltpu.SemaphoreType.DMA,
        pltpu.SemaphoreType.DMA,
    ],
)


def _mm_scale_body(x_ref, w_ref, degp_ref, g_ref, dinv_ref):
    h = jnp.dot(x_ref[...], w_ref[...], preferred_element_type=jnp.float32)
    degp = degp_ref[...]
    deg = degp[:, 0:1] + degp[:, 8:9] + 1.0
    dinv = lax.rsqrt(deg)
    g_ref[...] = h * dinv
    dinv_ref[...] = jnp.broadcast_to(dinv, (_BLK, 8))


_mm_scale = pl.pallas_call(
    _mm_scale_body,
    grid=(_GRID,),
    in_specs=[
        pl.BlockSpec((_BLK, _D), lambda i: (i, 0)),
        pl.BlockSpec((_D, _H), lambda i: (0, 0)),
        pl.BlockSpec((_BLK, _H), lambda i: (i, 0)),
    ],
    out_specs=[
        pl.BlockSpec((_BLK, _H), lambda i: (i, 0)),
        pl.BlockSpec((_BLK, 8), lambda i: (i, 0)),
    ],
    out_shape=[
        jax.ShapeDtypeStruct((_N, _H), jnp.float32),
        jax.ShapeDtypeStruct((_N, 8), jnp.float32),
    ],
)


def _bn_fused_body(z1p_ref, dinv_ref, gamma_ref, beta_ref, w2_ref,
                   g2_ref, y1_vmem, s1_ref, s2_ref):
    p = pl.program_id(0)
    i = pl.program_id(1)

    @pl.when(p == 0)
    def _stats_phase():
        dinv = dinv_ref[...][:, 0:1]
        y1 = dinv * z1p_ref[...]
        y1_vmem[pl.ds(i * _BLK, _BLK), :] = y1
        s1 = jnp.sum(y1, axis=0, keepdims=True)
        s2 = jnp.sum(y1 * y1, axis=0, keepdims=True)

        @pl.when(i == 0)
        def _init():
            s1_ref[...] = s1
            s2_ref[...] = s2

        @pl.when(i != 0)
        def _acc():
            s1_ref[...] += s1
            s2_ref[...] += s2

    @pl.when(p == 1)
    def _apply_phase():
        mean = s1_ref[...] / _N
        var = s2_ref[...] / _N - mean * mean
        scale = lax.rsqrt(var + 1e-5) * gamma_ref[...]
        shift = beta_ref[...] - mean * scale
        y1 = y1_vmem[pl.ds(i * _BLK, _BLK), :]
        x2 = jnp.maximum(y1 * scale + shift, 0.0)
        y = jnp.dot(x2, w2_ref[...], preferred_element_type=jnp.float32)
        g2_ref[...] = jnp.broadcast_to(dinv_ref[...][:, 0:1] * y,
                                       (_BLK, _H))


_bn_fused = pl.pallas_call(
    _bn_fused_body,
    grid=(2, _GRID),
    in_specs=[
        pl.BlockSpec((_BLK, _H), lambda p, i: (i * (1 - p), 0)),
        pl.BlockSpec((_BLK, 8), lambda p, i: (i, 0)),
        pl.BlockSpec((1, _H), lambda p, i: (0, 0)),
        pl.BlockSpec((1, _H), lambda p, i: (0, 0)),
        pl.BlockSpec((_H, 1), lambda p, i: (0, 0)),
    ],
    out_specs=pl.BlockSpec((_BLK, _H), lambda p, i: (i, 0)),
    out_shape=jax.ShapeDtypeStruct((_N, _H), jnp.float32),
    scratch_shapes=[
        pltpu.VMEM((_N, _H), jnp.float32),
        pltpu.VMEM((1, _H), jnp.float32),
        pltpu.VMEM((1, _H), jnp.float32),
    ],
)


def _out_body(z2p_ref, dinv_ref, b2_ref, o_ref):
    dinv = dinv_ref[...][:, 0:1]
    z2p = z2p_ref[...]
    z2 = z2p[:, 0:1] + z2p[:, 8:9]
    o_ref[...] = dinv * z2 + b2_ref[0, 0]


_out_k = pl.pallas_call(
    _out_body,
    grid=(_GRID,),
    in_specs=[
        pl.BlockSpec((_BLK, _H), lambda i: (i, 0)),
        pl.BlockSpec((_BLK, 8), lambda i: (i, 0)),
        pl.BlockSpec((1, 1), lambda i: (0, 0)),
    ],
    out_specs=pl.BlockSpec((_BLK, 1), lambda i: (i, 0)),
    out_shape=jax.ShapeDtypeStruct((_N, 1), jnp.float32),
)


def kernel(features, edge_index, W1, b1, gamma, beta, W2, b2):
    del b1
    zeros8 = jnp.zeros((_N, 8), jnp.float32)
    ones8 = jnp.ones((_KA, 8), jnp.float32)

    degp = _deg_kernel(edge_index, ones8, zeros8)
    g, dinv8 = _mm_scale(features, W1, degp)
    z1 = _agg64(edge_index, g)
    g2 = _bn_fused(z1, dinv8, gamma.reshape(1, _H),
                   beta.reshape(1, _H), W2)
    z2p = _agg8(edge_index, g2, zeros8)
    return _out_k(z2p, dinv8, b2.reshape(1, 1))
